# baseline (speedup 1.0000x reference)
import functools

import jax
import jax.numpy as jnp
from jax.experimental import pallas as pl
from jax.experimental.pallas import tpu as pltpu

NQ = 8
NE = 1024
D = 256


def _rvq_kernel(x_ref, emb_ref, embt_ref, q_ref, codes_ref, loss_ref,
                ehi_ref, emid_ref, elo_ref, en_ref, *, blk, n_total):
    @pl.when(pl.program_id(0) == 0)
    def _init():
        for i in range(NQ):
            emb_i = emb_ref[i]
            e_hi = emb_i.astype(jnp.bfloat16)
            rem1 = emb_i - e_hi.astype(jnp.float32)
            e_mid = rem1.astype(jnp.bfloat16)
            rem2 = rem1 - e_mid.astype(jnp.float32)
            ehi_ref[i] = e_hi
            emid_ref[i] = e_mid
            elo_ref[i] = rem2.astype(jnp.bfloat16)
            en_ref[i] = jnp.sum(emb_i * emb_i, axis=1,
                                keepdims=True).reshape(1, NE)

    r = x_ref[...]
    qsum = jnp.zeros_like(r)
    loss = jnp.float32(0.0)
    iota = jax.lax.broadcasted_iota(jnp.int32, (blk, NE), 1)
    idx_cols = []
    for i in range(NQ):
        zn = jnp.sum(r * r, axis=1, keepdims=True)
        en = en_ref[i]
        mm = jnp.dot(r.astype(jnp.bfloat16), embt_ref[i],
                     preferred_element_type=jnp.float32)
        dist = (zn + en) - 2.0 * mm
        idxc = jnp.argmin(dist, axis=1).reshape(blk, 1).astype(jnp.int32)
        oh = (iota == idxc).astype(jnp.bfloat16)
        q = (jnp.dot(oh, ehi_ref[i], preferred_element_type=jnp.float32)
             + jnp.dot(oh, emid_ref[i], preferred_element_type=jnp.float32)
             + jnp.dot(oh, elo_ref[i], preferred_element_type=jnp.float32))
        loss += 0.25 * (jnp.sum((q - r) ** 2) / jnp.float32(n_total))
        xq = r + (q - r)
        qsum = qsum + xq
        r = r - xq
        idx_cols.append(idxc)

    q_ref[...] = qsum
    codes_ref[...] = jnp.concatenate(idx_cols, axis=1)
    loss_ref[...] = jnp.full((1, 1, 1), loss, jnp.float32)


def kernel(x, emb):
    B, T, Dd = x.shape
    n = B * T
    flat = x.reshape(n, Dd)
    embt = jnp.swapaxes(emb, 1, 2).astype(jnp.bfloat16)
    blk = 1152
    nblk = n // blk

    q, codes, loss = pl.pallas_call(
        functools.partial(_rvq_kernel, blk=blk, n_total=n * Dd),
        grid=(nblk,),
        in_specs=[
            pl.BlockSpec((blk, Dd), lambda i: (i, 0)),
            pl.BlockSpec((NQ, NE, Dd), lambda i: (0, 0, 0)),
            pl.BlockSpec((NQ, Dd, NE), lambda i: (0, 0, 0)),
        ],
        out_specs=[
            pl.BlockSpec((blk, Dd), lambda i: (i, 0)),
            pl.BlockSpec((blk, NQ), lambda i: (i, 0)),
            pl.BlockSpec((1, 1, 1), lambda i: (i, 0, 0)),
        ],
        out_shape=[
            jax.ShapeDtypeStruct((n, Dd), jnp.float32),
            jax.ShapeDtypeStruct((n, NQ), jnp.int32),
            jax.ShapeDtypeStruct((nblk, 1, 1), jnp.float32),
        ],
        scratch_shapes=[
            pltpu.VMEM((NQ, NE, Dd), jnp.bfloat16),
            pltpu.VMEM((NQ, NE, Dd), jnp.bfloat16),
            pltpu.VMEM((NQ, NE, Dd), jnp.bfloat16),
            pltpu.VMEM((NQ, 1, NE), jnp.float32),
        ],
        compiler_params=pltpu.CompilerParams(
            dimension_semantics=("arbitrary",),
        ),
    )(flat, emb, embt)

    return q.reshape(B, T, Dd), jnp.sum(loss), codes.reshape(B, T, NQ)


# speedup vs baseline: 1.3169x; 1.3169x over previous
import functools

import jax
import jax.numpy as jnp
from jax.experimental import pallas as pl
from jax.experimental.pallas import tpu as pltpu

NQ = 8
NE = 1024
D = 256


def _rvq_kernel(x_ref, emb_ref, embt_ref, q_ref, codes_ref, loss_ref, *,
                blk, n_total):
    r = x_ref[...]
    qsum = jnp.zeros_like(r)
    loss = jnp.float32(0.0)
    iota = jax.lax.broadcasted_iota(jnp.int32, (blk, NE), 1)
    idx_cols = []
    for i in range(NQ):
        emb_i = emb_ref[i]
        zn = jnp.sum(r * r, axis=1, keepdims=True)
        en = jnp.sum(emb_i * emb_i, axis=1, keepdims=True).reshape(1, NE)
        mm = jnp.dot(r.astype(jnp.bfloat16), embt_ref[i],
                     preferred_element_type=jnp.float32)
        dist = (zn + en) - 2.0 * mm
        idxc = jnp.argmin(dist, axis=1).reshape(blk, 1).astype(jnp.int32)
        oh = (iota == idxc).astype(jnp.bfloat16)
        e_hi = emb_i.astype(jnp.bfloat16)
        rem1 = emb_i - e_hi.astype(jnp.float32)
        e_mid = rem1.astype(jnp.bfloat16)
        rem2 = rem1 - e_mid.astype(jnp.float32)
        e_lo = rem2.astype(jnp.bfloat16)
        q = (jnp.dot(oh, e_hi, preferred_element_type=jnp.float32)
             + jnp.dot(oh, e_mid, preferred_element_type=jnp.float32)
             + jnp.dot(oh, e_lo, preferred_element_type=jnp.float32))
        loss += 0.25 * (jnp.sum((q - r) ** 2) / jnp.float32(n_total))
        xq = r + (q - r)
        qsum = qsum + xq
        r = r - xq
        idx_cols.append(idxc)

    q_ref[...] = qsum
    codes_ref[...] = jnp.concatenate(idx_cols, axis=1)
    loss_ref[...] = jnp.full((1, 1, 1), loss, jnp.float32)


def kernel(x, emb):
    B, T, Dd = x.shape
    n = B * T
    flat = x.reshape(n, Dd)
    embt = jnp.swapaxes(emb, 1, 2).astype(jnp.bfloat16)
    blk = 1152
    nblk = n // blk

    q, codes, loss = pl.pallas_call(
        functools.partial(_rvq_kernel, blk=blk, n_total=n * Dd),
        grid=(nblk,),
        in_specs=[
            pl.BlockSpec((blk, Dd), lambda i: (i, 0)),
            pl.BlockSpec((NQ, NE, Dd), lambda i: (0, 0, 0)),
            pl.BlockSpec((NQ, Dd, NE), lambda i: (0, 0, 0)),
        ],
        out_specs=[
            pl.BlockSpec((blk, Dd), lambda i: (i, 0)),
            pl.BlockSpec((blk, NQ), lambda i: (i, 0)),
            pl.BlockSpec((1, 1, 1), lambda i: (i, 0, 0)),
        ],
        out_shape=[
            jax.ShapeDtypeStruct((n, Dd), jnp.float32),
            jax.ShapeDtypeStruct((n, NQ), jnp.int32),
            jax.ShapeDtypeStruct((nblk, 1, 1), jnp.float32),
        ],
        compiler_params=pltpu.CompilerParams(
            dimension_semantics=("parallel",),
        ),
    )(flat, emb, embt)

    return q.reshape(B, T, Dd), jnp.sum(loss), codes.reshape(B, T, NQ)
